# Initial kernel scaffold; baseline (speedup 1.0000x reference)
import jax
import jax.numpy as jnp
from jax.experimental import pallas as pl


def kernel(hidden_states, W, b):
    raise NotImplementedError("write your pallas kernel here")


# speedup vs baseline: 1.2005x; 1.2005x over previous
import functools

import jax
import jax.numpy as jnp
from jax import lax
from jax.experimental import pallas as pl
from jax.experimental.pallas import tpu as pltpu
from jax.experimental.pallas import tpu_sc as plsc

_E = 64
_K = 2
_T = 512


def _logits_body(x_ref, w_ref, b_ref, out_ref):
    acc = lax.dot_general(
        w_ref[...], x_ref[...],
        (((1,), (1,)), ((), ())),
        preferred_element_type=jnp.float32,
    )
    out_ref[...] = acc + b_ref[...]


def _tc_logits(x, W, b2d):
    ntok, H = x.shape
    grid = ntok // _T
    return pl.pallas_call(
        _logits_body,
        grid=(grid,),
        in_specs=[
            pl.BlockSpec((_T, H), lambda i: (i, 0)),
            pl.BlockSpec((_E, H), lambda i: (0, 0)),
            pl.BlockSpec((_E, 1), lambda i: (0, 0)),
        ],
        out_specs=pl.BlockSpec((_E, _T), lambda i: (0, i)),
        out_shape=jax.ShapeDtypeStruct((_E, ntok), jnp.float32),
    )(x, W, b2d)


@functools.cache
def _sc_router(ntok):
    info = plsc.get_sparse_core_info()
    nc, ns, L = info.num_cores, info.num_subcores, info.num_lanes
    nw = nc * ns
    tpw = ntok // nw
    mesh = plsc.VectorSubcoreMesh(core_axis_name="c", subcore_axis_name="s")

    @functools.partial(
        pl.kernel,
        out_type=(
            jax.ShapeDtypeStruct((_K, ntok), jnp.int32),
            jax.ShapeDtypeStruct((_K, ntok), jnp.float32),
        ),
        mesh=mesh,
        scratch_types=[
            pltpu.VMEM((_E, tpw), jnp.float32),
            pltpu.VMEM((_K, tpw), jnp.int32),
            pltpu.VMEM((_K, tpw), jnp.float32),
        ],
    )
    def route(logits_hbm, idx_hbm, w_hbm, chunk_v, idx_v, w_v):
        wid = lax.axis_index("s") * nc + lax.axis_index("c")
        base = wid * tpw
        pltpu.sync_copy(logits_hbm.at[:, pl.ds(base, tpw)], chunk_v)

        def group(g, carry):
            off = g * L
            neg = jnp.full((L,), -jnp.inf, jnp.float32)
            zero = jnp.zeros((L,), jnp.int32)
            m1, m2, i1, i2 = neg, neg, zero, zero
            for e in range(_E):
                v = chunk_v[e, pl.ds(off, L)]
                gt1 = v > m1
                gt2 = v > m2
                ev = jnp.full((L,), e, jnp.int32)
                m2 = jnp.where(gt1, m1, jnp.where(gt2, v, m2))
                i2 = jnp.where(gt1, i1, jnp.where(gt2, ev, i2))
                m1 = jnp.where(gt1, v, m1)
                i1 = jnp.where(gt1, ev, i1)
            w1 = 1.0 / (1.0 + jnp.exp(m2 - m1))
            idx_v[0, pl.ds(off, L)] = i1
            idx_v[1, pl.ds(off, L)] = i2
            w_v[0, pl.ds(off, L)] = w1
            w_v[1, pl.ds(off, L)] = 1.0 - w1
            return carry

        lax.fori_loop(0, tpw // L, group, 0)
        pltpu.sync_copy(idx_v, idx_hbm.at[:, pl.ds(base, tpw)])
        pltpu.sync_copy(w_v, w_hbm.at[:, pl.ds(base, tpw)])

    return route


def kernel(hidden_states, W, b):
    B, S, H = hidden_states.shape
    ntok = B * S
    x = hidden_states.reshape(ntok, H)
    logits_t = _tc_logits(x, W, b.reshape(_E, 1))
    idx_t, w_t = _sc_router(ntok)(logits_t)
    expert_indices = idx_t.T.reshape(B, S, _K)
    routing_weights = w_t.T.reshape(B, S, _K)
    return expert_indices, routing_weights


# speedup vs baseline: 1.2985x; 1.0817x over previous
import functools

import jax
import jax.numpy as jnp
from jax import lax
from jax.experimental import pallas as pl
from jax.experimental.pallas import tpu as pltpu
from jax.experimental.pallas import tpu_sc as plsc

_E = 64
_K = 2
_T = 1024


def _logits_body(x_ref, w_ref, b_ref, out_ref):
    acc = lax.dot_general(
        w_ref[...], x_ref[...],
        (((1,), (1,)), ((), ())),
        preferred_element_type=jnp.float32,
    )
    out_ref[...] = acc + b_ref[...]


def _tc_logits(x, W, b2d):
    ntok, H = x.shape
    grid = ntok // _T
    return pl.pallas_call(
        _logits_body,
        grid=(grid,),
        in_specs=[
            pl.BlockSpec((_T, H), lambda i: (i, 0)),
            pl.BlockSpec((_E, H), lambda i: (0, 0)),
            pl.BlockSpec((_E, 1), lambda i: (0, 0)),
        ],
        out_specs=pl.BlockSpec((_E, _T), lambda i: (0, i)),
        out_shape=jax.ShapeDtypeStruct((_E, ntok), jnp.float32),
    )(x, W, b2d)


@functools.cache
def _sc_router(ntok):
    info = plsc.get_sparse_core_info()
    nc, ns, L = info.num_cores, info.num_subcores, info.num_lanes
    nw = nc * ns
    tpw = ntok // nw
    mesh = plsc.VectorSubcoreMesh(core_axis_name="c", subcore_axis_name="s")

    @functools.partial(
        pl.kernel,
        out_type=(
            jax.ShapeDtypeStruct((_K, ntok), jnp.int32),
            jax.ShapeDtypeStruct((_K, ntok), jnp.float32),
        ),
        mesh=mesh,
        scratch_types=[
            pltpu.VMEM((_E, tpw), jnp.float32),
            pltpu.VMEM((_K, tpw), jnp.int32),
            pltpu.VMEM((_K, tpw), jnp.float32),
        ],
    )
    def route(logits_hbm, idx_hbm, w_hbm, chunk_v, idx_v, w_v):
        wid = lax.axis_index("s") * nc + lax.axis_index("c")
        base = wid * tpw
        pltpu.sync_copy(logits_hbm.at[:, pl.ds(base, tpw)], chunk_v)

        def group(g, carry):
            off = g * L
            neg = jnp.full((L,), -jnp.inf, jnp.float32)
            zero = jnp.zeros((L,), jnp.int32)
            m1, m2, i1, i2 = neg, neg, zero, zero
            for e in range(_E):
                v = chunk_v[e, pl.ds(off, L)]
                gt1 = v > m1
                gt2 = v > m2
                ev = jnp.full((L,), e, jnp.int32)
                m2 = jnp.where(gt1, m1, jnp.where(gt2, v, m2))
                i2 = jnp.where(gt1, i1, jnp.where(gt2, ev, i2))
                m1 = jnp.where(gt1, v, m1)
                i1 = jnp.where(gt1, ev, i1)
            w1 = 1.0 / (1.0 + jnp.exp(m2 - m1))
            idx_v[0, pl.ds(off, L)] = i1
            idx_v[1, pl.ds(off, L)] = i2
            w_v[0, pl.ds(off, L)] = w1
            w_v[1, pl.ds(off, L)] = 1.0 - w1
            return carry

        lax.fori_loop(0, tpw // L, group, 0)
        pltpu.sync_copy(idx_v, idx_hbm.at[:, pl.ds(base, tpw)])
        pltpu.sync_copy(w_v, w_hbm.at[:, pl.ds(base, tpw)])

    return route


def kernel(hidden_states, W, b):
    B, S, H = hidden_states.shape
    ntok = B * S
    x = hidden_states.reshape(ntok, H)
    logits_t = _tc_logits(x, W, b.reshape(_E, 1))
    idx_t, w_t = _sc_router(ntok)(logits_t)
    expert_indices = idx_t.T.reshape(B, S, _K)
    routing_weights = w_t.T.reshape(B, S, _K)
    return expert_indices, routing_weights
